# baseline (speedup 1.0000x reference)
import functools

import jax
import jax.numpy as jnp
from jax import lax
from jax.experimental import pallas as pl
from jax.experimental.pallas import tpu as pltpu
from jax.experimental.pallas import tpu_sc as plsc

NC = 2
NS = 16
NW = NC * NS
CHUNK = 128
GPB = 2
OCHUNK = GPB * CHUNK
NBUF = 3


def _gather_body(nb, b, feat_hbm, idx_hbm, out_hbm, idx_v, *scratch):
    bufs = scratch[:NBUF]
    gsems = scratch[NBUF:2 * NBUF]
    osems = scratch[2 * NBUF:3 * NBUF]

    rows_per_w = nb * OCHUNK
    last_rows = b - (NW - 1) * rows_per_w
    last_full = last_rows // OCHUNK
    tail = last_rows - last_full * OCHUNK

    wid = lax.axis_index("s") * NC + lax.axis_index("c")
    out_base = wid * rows_per_w
    is_last = wid == NW - 1

    pltpu.sync_copy(idx_hbm.at[wid], idx_v)

    def start_gathers(c, j):
        for h in range(GPB):
            dst = bufs[j].at[pl.ds(h * CHUNK, CHUNK)]
            pltpu.make_async_copy(
                feat_hbm.at[idx_v.at[GPB * c + h]], dst, gsems[j]).start()

    def wait_gathers(j):
        for h in range(GPB):
            dst = bufs[j].at[pl.ds(h * CHUNK, CHUNK)]
            pltpu.make_async_copy(feat_hbm.at[idx_v.at[0]], dst, gsems[j]).wait()

    def full_out(c, j):
        dst = out_hbm.at[pl.ds(out_base + c * OCHUNK, OCHUNK)]
        return pltpu.make_async_copy(bufs[j], dst, osems[j])

    def tail_out(j):
        dst = out_hbm.at[pl.ds((NW - 1) * rows_per_w + last_full * OCHUNK, tail)]
        return pltpu.make_async_copy(bufs[j].at[pl.ds(0, tail)], dst, osems[j])

    def start_out(c, j):
        @pl.when(jnp.logical_or(jnp.logical_not(is_last), c < last_full))
        def _():
            full_out(c, j).start()

        if tail > 0:
            @pl.when(jnp.logical_and(is_last, c == last_full))
            def _():
                tail_out(j).start()

    def wait_out(c, j):
        @pl.when(jnp.logical_or(jnp.logical_not(is_last), c < last_full))
        def _():
            full_out(0, j).wait()

        if tail > 0:
            @pl.when(jnp.logical_and(is_last, c == last_full))
            def _():
                tail_out(j).wait()

    for c in range(min(NBUF - 1, nb)):
        start_gathers(c, c)

    def step(c, j):
        wait_gathers(j)
        start_out(c, j)

    def feed(c, f, k2, guard_prev):
        if guard_prev:
            wait_out(c - 1, k2)
        start_gathers(f, k2)

    head_end = min(NBUF, nb)
    for c in range(head_end):
        step(c, c % NBUF)
        f = c + NBUF - 1
        if f < nb:
            feed(c, f, f % NBUF, c >= 1)

    m_end = ((nb - (NBUF - 1)) // NBUF) * NBUF if nb > NBUF else head_end
    m_end = max(m_end, head_end)

    def grp(g, _):
        base = NBUF + g * NBUF
        for j in range(NBUF):
            c = base + j
            step(c, j)
            k2 = (j + NBUF - 1) % NBUF
            wait_out(c - 1, k2)
            start_gathers(c + NBUF - 1, k2)
        return 0

    if m_end > head_end:
        lax.fori_loop(0, (m_end - NBUF) // NBUF, grp, 0)

    for c in range(m_end, nb):
        step(c, c % NBUF)
        f = c + NBUF - 1
        if f < nb:
            feed(c, f, f % NBUF, True)

    for c in range(max(0, nb - NBUF), nb):
        wait_out(c, c % NBUF)


@functools.partial(jax.jit, static_argnames=("nb", "b"))
def _gather(features, idx3d, nb, b):
    d = features.shape[1]
    n_chunks = idx3d.shape[1]
    mesh = plsc.VectorSubcoreMesh(
        core_axis_name="c", subcore_axis_name="s",
        num_cores=NC, num_subcores=NS)
    return pl.kernel(
        functools.partial(_gather_body, nb, b),
        out_type=jax.ShapeDtypeStruct((b, d), features.dtype),
        mesh=mesh,
        scratch_types=(
            [pltpu.VMEM((n_chunks, CHUNK), jnp.int32)]
            + [pltpu.VMEM((OCHUNK, d), features.dtype) for _ in range(NBUF)]
            + [pltpu.SemaphoreType.DMA for _ in range(2 * NBUF)]
        ),
    )(features, idx3d)


def kernel(features, rev):
    b = rev.shape[0]
    n_rows = features.shape[0]
    rev = rev.astype(jnp.int32)
    unit = NW * OCHUNK
    b_pad = ((b + unit - 1) // unit) * unit
    nb = b_pad // (NW * OCHUNK)
    if b_pad != b:
        pad = jnp.arange(b_pad - b, dtype=jnp.int32) % n_rows
        rev = jnp.concatenate([rev, pad])
    idx3d = rev.reshape(NW, nb * GPB, CHUNK)
    return _gather(features, idx3d, nb, b)


# speedup vs baseline: 1.0371x; 1.0371x over previous
import functools

import jax
import jax.numpy as jnp
from jax import lax
from jax.experimental import pallas as pl
from jax.experimental.pallas import tpu as pltpu
from jax.experimental.pallas import tpu_sc as plsc

NC = 2
NS = 16
NW = NC * NS
CHUNK = 128
GPB = 1
OCHUNK = GPB * CHUNK
NBUF = 6


def _gather_body(nb, b, feat_hbm, idx_hbm, out_hbm, idx_v, *scratch):
    bufs = scratch[:NBUF]
    gsems = scratch[NBUF:2 * NBUF]
    osems = scratch[2 * NBUF:3 * NBUF]

    rows_per_w = nb * OCHUNK
    last_rows = b - (NW - 1) * rows_per_w
    last_full = last_rows // OCHUNK
    tail = last_rows - last_full * OCHUNK

    wid = lax.axis_index("s") * NC + lax.axis_index("c")
    out_base = wid * rows_per_w
    is_last = wid == NW - 1

    pltpu.sync_copy(idx_hbm.at[wid], idx_v)

    def start_gathers(c, j):
        for h in range(GPB):
            dst = bufs[j].at[pl.ds(h * CHUNK, CHUNK)]
            pltpu.make_async_copy(
                feat_hbm.at[idx_v.at[GPB * c + h]], dst, gsems[j]).start()

    def wait_gathers(j):
        for h in range(GPB):
            dst = bufs[j].at[pl.ds(h * CHUNK, CHUNK)]
            pltpu.make_async_copy(feat_hbm.at[idx_v.at[0]], dst, gsems[j]).wait()

    def full_out(c, j):
        dst = out_hbm.at[pl.ds(out_base + c * OCHUNK, OCHUNK)]
        return pltpu.make_async_copy(bufs[j], dst, osems[j])

    def tail_out(j):
        dst = out_hbm.at[pl.ds((NW - 1) * rows_per_w + last_full * OCHUNK, tail)]
        return pltpu.make_async_copy(bufs[j].at[pl.ds(0, tail)], dst, osems[j])

    def start_out(c, j):
        @pl.when(jnp.logical_or(jnp.logical_not(is_last), c < last_full))
        def _():
            full_out(c, j).start()

        if tail > 0:
            @pl.when(jnp.logical_and(is_last, c == last_full))
            def _():
                tail_out(j).start()

    def wait_out(c, j):
        @pl.when(jnp.logical_or(jnp.logical_not(is_last), c < last_full))
        def _():
            full_out(0, j).wait()

        if tail > 0:
            @pl.when(jnp.logical_and(is_last, c == last_full))
            def _():
                tail_out(j).wait()

    for c in range(min(NBUF - 1, nb)):
        start_gathers(c, c)

    def step(c, j):
        wait_gathers(j)
        start_out(c, j)

    def feed(c, f, k2, guard_prev):
        if guard_prev:
            wait_out(c - 1, k2)
        start_gathers(f, k2)

    head_end = min(NBUF, nb)
    for c in range(head_end):
        step(c, c % NBUF)
        f = c + NBUF - 1
        if f < nb:
            feed(c, f, f % NBUF, c >= 1)

    m_end = ((nb - (NBUF - 1)) // NBUF) * NBUF if nb > NBUF else head_end
    m_end = max(m_end, head_end)

    def grp(g, _):
        base = NBUF + g * NBUF
        for j in range(NBUF):
            c = base + j
            step(c, j)
            k2 = (j + NBUF - 1) % NBUF
            wait_out(c - 1, k2)
            start_gathers(c + NBUF - 1, k2)
        return 0

    if m_end > head_end:
        lax.fori_loop(0, (m_end - NBUF) // NBUF, grp, 0)

    for c in range(m_end, nb):
        step(c, c % NBUF)
        f = c + NBUF - 1
        if f < nb:
            feed(c, f, f % NBUF, True)

    for c in range(max(0, nb - NBUF), nb):
        wait_out(c, c % NBUF)


@functools.partial(jax.jit, static_argnames=("nb", "b"))
def _gather(features, idx3d, nb, b):
    d = features.shape[1]
    n_chunks = idx3d.shape[1]
    mesh = plsc.VectorSubcoreMesh(
        core_axis_name="c", subcore_axis_name="s",
        num_cores=NC, num_subcores=NS)
    return pl.kernel(
        functools.partial(_gather_body, nb, b),
        out_type=jax.ShapeDtypeStruct((b, d), features.dtype),
        mesh=mesh,
        scratch_types=(
            [pltpu.VMEM((n_chunks, CHUNK), jnp.int32)]
            + [pltpu.VMEM((OCHUNK, d), features.dtype) for _ in range(NBUF)]
            + [pltpu.SemaphoreType.DMA for _ in range(2 * NBUF)]
        ),
    )(features, idx3d)


def kernel(features, rev):
    b = rev.shape[0]
    n_rows = features.shape[0]
    rev = rev.astype(jnp.int32)
    unit = NW * OCHUNK
    b_pad = ((b + unit - 1) // unit) * unit
    nb = b_pad // (NW * OCHUNK)
    if b_pad != b:
        pad = jnp.arange(b_pad - b, dtype=jnp.int32) % n_rows
        rev = jnp.concatenate([rev, pad])
    idx3d = rev.reshape(NW, nb * GPB, CHUNK)
    return _gather(features, idx3d, nb, b)


# speedup vs baseline: 1.0429x; 1.0057x over previous
import functools

import jax
import jax.numpy as jnp
from jax import lax
from jax.experimental import pallas as pl
from jax.experimental.pallas import tpu as pltpu
from jax.experimental.pallas import tpu_sc as plsc

NC = 2
NS = 16
NW = NC * NS
CHUNK = 128
GPB = 1
OCHUNK = GPB * CHUNK
NBUF = 7


def _gather_body(nb, b, feat_hbm, idx_hbm, out_hbm, idx_v, *scratch):
    bufs = scratch[:NBUF]
    gsems = scratch[NBUF:2 * NBUF]
    osems = scratch[2 * NBUF:3 * NBUF]

    rows_per_w = nb * OCHUNK
    last_rows = b - (NW - 1) * rows_per_w
    last_full = last_rows // OCHUNK
    tail = last_rows - last_full * OCHUNK

    wid = lax.axis_index("s") * NC + lax.axis_index("c")
    out_base = wid * rows_per_w
    is_last = wid == NW - 1

    pltpu.sync_copy(idx_hbm.at[wid], idx_v)

    def start_gathers(c, j):
        for h in range(GPB):
            dst = bufs[j].at[pl.ds(h * CHUNK, CHUNK)]
            pltpu.make_async_copy(
                feat_hbm.at[idx_v.at[GPB * c + h]], dst, gsems[j]).start()

    def wait_gathers(j):
        for h in range(GPB):
            dst = bufs[j].at[pl.ds(h * CHUNK, CHUNK)]
            pltpu.make_async_copy(feat_hbm.at[idx_v.at[0]], dst, gsems[j]).wait()

    def full_out(c, j):
        dst = out_hbm.at[pl.ds(out_base + c * OCHUNK, OCHUNK)]
        return pltpu.make_async_copy(bufs[j], dst, osems[j])

    def tail_out(j):
        dst = out_hbm.at[pl.ds((NW - 1) * rows_per_w + last_full * OCHUNK, tail)]
        return pltpu.make_async_copy(bufs[j].at[pl.ds(0, tail)], dst, osems[j])

    def start_out(c, j):
        @pl.when(jnp.logical_or(jnp.logical_not(is_last), c < last_full))
        def _():
            full_out(c, j).start()

        if tail > 0:
            @pl.when(jnp.logical_and(is_last, c == last_full))
            def _():
                tail_out(j).start()

    def wait_out(c, j):
        @pl.when(jnp.logical_or(jnp.logical_not(is_last), c < last_full))
        def _():
            full_out(0, j).wait()

        if tail > 0:
            @pl.when(jnp.logical_and(is_last, c == last_full))
            def _():
                tail_out(j).wait()

    for c in range(min(NBUF - 1, nb)):
        start_gathers(c, c)

    def step(c, j):
        wait_gathers(j)
        start_out(c, j)

    def feed(c, f, k2, guard_prev):
        if guard_prev:
            wait_out(c - 1, k2)
        start_gathers(f, k2)

    head_end = min(NBUF, nb)
    for c in range(head_end):
        step(c, c % NBUF)
        f = c + NBUF - 1
        if f < nb:
            feed(c, f, f % NBUF, c >= 1)

    m_end = ((nb - (NBUF - 1)) // NBUF) * NBUF if nb > NBUF else head_end
    m_end = max(m_end, head_end)

    def grp(g, _):
        base = NBUF + g * NBUF
        for j in range(NBUF):
            c = base + j
            step(c, j)
            k2 = (j + NBUF - 1) % NBUF
            wait_out(c - 1, k2)
            start_gathers(c + NBUF - 1, k2)
        return 0

    if m_end > head_end:
        lax.fori_loop(0, (m_end - NBUF) // NBUF, grp, 0)

    for c in range(m_end, nb):
        step(c, c % NBUF)
        f = c + NBUF - 1
        if f < nb:
            feed(c, f, f % NBUF, True)

    for c in range(max(0, nb - NBUF), nb):
        wait_out(c, c % NBUF)


@functools.partial(jax.jit, static_argnames=("nb", "b"))
def _gather(features, idx3d, nb, b):
    d = features.shape[1]
    n_chunks = idx3d.shape[1]
    mesh = plsc.VectorSubcoreMesh(
        core_axis_name="c", subcore_axis_name="s",
        num_cores=NC, num_subcores=NS)
    return pl.kernel(
        functools.partial(_gather_body, nb, b),
        out_type=jax.ShapeDtypeStruct((b, d), features.dtype),
        mesh=mesh,
        scratch_types=(
            [pltpu.VMEM((n_chunks, CHUNK), jnp.int32)]
            + [pltpu.VMEM((OCHUNK, d), features.dtype) for _ in range(NBUF)]
            + [pltpu.SemaphoreType.DMA for _ in range(2 * NBUF)]
        ),
    )(features, idx3d)


def kernel(features, rev):
    b = rev.shape[0]
    n_rows = features.shape[0]
    rev = rev.astype(jnp.int32)
    unit = NW * OCHUNK
    b_pad = ((b + unit - 1) // unit) * unit
    nb = b_pad // (NW * OCHUNK)
    if b_pad != b:
        pad = jnp.arange(b_pad - b, dtype=jnp.int32) % n_rows
        rev = jnp.concatenate([rev, pad])
    idx3d = rev.reshape(NW, nb * GPB, CHUNK)
    return _gather(features, idx3d, nb, b)
